# Initial kernel scaffold; baseline (speedup 1.0000x reference)
import functools

import jax
import jax.numpy as jnp
from jax import lax
from jax.experimental import pallas as pl
from jax.experimental.pallas import tpu as pltpu
from jax.experimental.pallas import tpu_sc as plsc

N_NODES = 10000
N_EDGES = 320000
D_NODE = 128
D_EDGE = 16
D_OUT = 128

NC = 2
NS = 16
NW = NC * NS
PER_W = N_EDGES // NW
CH = 80
NCH = PER_W // CH


def _node_proj_body(x_ref, w_ref, b_ref, p_ref, q_ref):
    x = x_ref[...]
    w1 = w_ref[0:D_NODE, :]
    w2 = w_ref[D_NODE:2 * D_NODE, :]
    p_ref[...] = jnp.dot(x, w1, preferred_element_type=jnp.float32) + b_ref[...]
    q_ref[...] = jnp.dot(x, w2, preferred_element_type=jnp.float32)


def _node_proj(x, W, b):
    return pl.pallas_call(
        _node_proj_body,
        out_shape=(
            jax.ShapeDtypeStruct((N_NODES, D_OUT), jnp.float32),
            jax.ShapeDtypeStruct((N_NODES, D_OUT), jnp.float32),
        ),
    )(x, W, b.reshape(1, D_OUT))


_EBLK = 8000


def _edge_proj_body(ea_ref, w_ref, e_ref):
    w3 = w_ref[2 * D_NODE:, :]
    e_ref[...] = jnp.dot(ea_ref[...], w3, preferred_element_type=jnp.float32)


def _edge_proj(edge_attr, W):
    grid = (N_EDGES // _EBLK,)
    return pl.pallas_call(
        _edge_proj_body,
        grid=grid,
        in_specs=[
            pl.BlockSpec((_EBLK, D_EDGE), lambda i: (i, 0)),
            pl.BlockSpec((W.shape[0], D_OUT), lambda i: (0, 0)),
        ],
        out_specs=pl.BlockSpec((_EBLK, D_OUT), lambda i: (i, 0)),
        out_shape=jax.ShapeDtypeStruct((N_EDGES, D_OUT), jnp.float32),
    )(edge_attr, W)


def _sc_body(ei_ref, p_ref, q_ref, e_ref, out_ref,
             idx_s, idx_d, pbuf, qbuf, ebuf, sem_p, sem_q):
    c = lax.axis_index("c")
    s = lax.axis_index("s")
    wid = s * NC + c
    base0 = wid * PER_W

    def chunk_body(i, carry):
        base = base0 + i * CH
        pltpu.sync_copy(ei_ref.at[0, pl.ds(base, CH)], idx_s)
        pltpu.sync_copy(ei_ref.at[1, pl.ds(base, CH)], idx_d)
        cp_p = pltpu.async_copy(p_ref.at[idx_s], pbuf, sem_p)
        cp_q = pltpu.async_copy(q_ref.at[idx_d], qbuf, sem_q)
        pltpu.sync_copy(e_ref.at[pl.ds(base, CH)], ebuf)
        cp_p.wait()
        cp_q.wait()

        def row_body(r, rcarry):
            for j in range(D_OUT // 16):
                sl = pl.ds(j * 16, 16)
                ebuf[r, sl] = ebuf[r, sl] + pbuf[r, sl] + qbuf[r, sl]
            return rcarry

        lax.fori_loop(0, CH, row_body, 0)
        pltpu.sync_copy(ebuf, out_ref.at[pl.ds(base, CH)])
        return carry

    lax.fori_loop(0, NCH, chunk_body, 0)


def _sc_gather_add(edge_index, P, Q, E):
    mesh = plsc.VectorSubcoreMesh(
        core_axis_name="c", subcore_axis_name="s", num_cores=NC, num_subcores=NS)
    k = functools.partial(
        pl.kernel,
        mesh=mesh,
        out_type=jax.ShapeDtypeStruct((N_EDGES, D_OUT), jnp.float32),
        scratch_types=[
            pltpu.VMEM((CH,), jnp.int32),
            pltpu.VMEM((CH,), jnp.int32),
            pltpu.VMEM((CH, D_OUT), jnp.float32),
            pltpu.VMEM((CH, D_OUT), jnp.float32),
            pltpu.VMEM((CH, D_OUT), jnp.float32),
            pltpu.SemaphoreType.DMA,
            pltpu.SemaphoreType.DMA,
        ],
    )(_sc_body)
    return k(edge_index, P, Q, E)


def kernel(x, edge_index, edge_attr, W, b):
    P, Q = _node_proj(x, W, b)
    E = _edge_proj(edge_attr, W)
    return _sc_gather_add(edge_index, P, Q, E)


# speedup vs baseline: 2.6257x; 2.6257x over previous
import functools

import jax
import jax.numpy as jnp
from jax import lax
from jax.experimental import pallas as pl
from jax.experimental.pallas import tpu as pltpu
from jax.experimental.pallas import tpu_sc as plsc

N_NODES = 10000
N_EDGES = 320000
D_NODE = 128
D_EDGE = 16
D_OUT = 128

NC = 2
NS = 16
NW = NC * NS
CH = 128
NCHUNKS = N_EDGES // CH
CH_PER_W = NCHUNKS // NW
CH_EXTRA = NCHUNKS - CH_PER_W * NW


def _node_proj_body(x_ref, w_ref, b_ref, p_ref, q_ref):
    x = x_ref[...]
    w1 = w_ref[0:D_NODE, :]
    w2 = w_ref[D_NODE:2 * D_NODE, :]
    p_ref[...] = jnp.dot(x, w1, preferred_element_type=jnp.float32) + b_ref[...]
    q_ref[...] = jnp.dot(x, w2, preferred_element_type=jnp.float32)


def _node_proj(x, W, b):
    return pl.pallas_call(
        _node_proj_body,
        out_shape=(
            jax.ShapeDtypeStruct((N_NODES, D_OUT), jnp.float32),
            jax.ShapeDtypeStruct((N_NODES, D_OUT), jnp.float32),
        ),
    )(x, W, b.reshape(1, D_OUT))


_EBLK = 8000


def _edge_proj_body(ea_ref, w_ref, e_ref):
    w3 = w_ref[2 * D_NODE:, :]
    e_ref[...] = jnp.dot(ea_ref[...], w3, preferred_element_type=jnp.float32)


def _edge_proj(edge_attr, W):
    grid = (N_EDGES // _EBLK,)
    return pl.pallas_call(
        _edge_proj_body,
        grid=grid,
        in_specs=[
            pl.BlockSpec((_EBLK, D_EDGE), lambda i: (i, 0)),
            pl.BlockSpec((W.shape[0], D_OUT), lambda i: (0, 0)),
        ],
        out_specs=pl.BlockSpec((_EBLK, D_OUT), lambda i: (i, 0)),
        out_shape=jax.ShapeDtypeStruct((N_EDGES, D_OUT), jnp.float32),
    )(edge_attr, W)


def _sc_body(src_ref, dst_ref, p_ref, q_ref, e_ref, out_ref,
             idx_s, idx_d, pbuf, qbuf, ebuf, sem_p, sem_q):
    c = lax.axis_index("c")
    s = lax.axis_index("s")
    wid = s * NC + c
    start = wid * CH_PER_W + jnp.minimum(wid, CH_EXTRA)
    nch = CH_PER_W + jnp.where(wid < CH_EXTRA, 1, 0)

    def chunk_body(i, carry):
        base = (start + i) * CH
        pltpu.sync_copy(src_ref.at[pl.ds(base, CH)], idx_s)
        pltpu.sync_copy(dst_ref.at[pl.ds(base, CH)], idx_d)
        cp_p = pltpu.async_copy(p_ref.at[idx_s], pbuf, sem_p)
        cp_q = pltpu.async_copy(q_ref.at[idx_d], qbuf, sem_q)
        pltpu.sync_copy(e_ref.at[pl.ds(base, CH)], ebuf)
        cp_p.wait()
        cp_q.wait()

        def row_body(r, rcarry):
            for j in range(D_OUT // 16):
                sl = pl.ds(j * 16, 16)
                ebuf[r, sl] = ebuf[r, sl] + pbuf[r, sl] + qbuf[r, sl]
            return rcarry

        lax.fori_loop(0, CH, row_body, 0)
        pltpu.sync_copy(ebuf, out_ref.at[pl.ds(base, CH)])
        return carry

    lax.fori_loop(0, nch, chunk_body, 0)


def _sc_gather_add(src, dst, P, Q, E):
    mesh = plsc.VectorSubcoreMesh(
        core_axis_name="c", subcore_axis_name="s", num_cores=NC, num_subcores=NS)
    k = functools.partial(
        pl.kernel,
        mesh=mesh,
        out_type=jax.ShapeDtypeStruct((N_EDGES, D_OUT), jnp.float32),
        scratch_types=[
            pltpu.VMEM((CH,), jnp.int32),
            pltpu.VMEM((CH,), jnp.int32),
            pltpu.VMEM((CH, D_OUT), jnp.float32),
            pltpu.VMEM((CH, D_OUT), jnp.float32),
            pltpu.VMEM((CH, D_OUT), jnp.float32),
            pltpu.SemaphoreType.DMA,
            pltpu.SemaphoreType.DMA,
        ],
    )(_sc_body)
    return k(src, dst, P, Q, E)


def kernel(x, edge_index, edge_attr, W, b):
    P, Q = _node_proj(x, W, b)
    E = _edge_proj(edge_attr, W)
    return _sc_gather_add(edge_index[0], edge_index[1], P, Q, E)


# speedup vs baseline: 3.4972x; 1.3319x over previous
import functools

import jax
import jax.numpy as jnp
from jax import lax
from jax.experimental import pallas as pl
from jax.experimental.pallas import tpu as pltpu
from jax.experimental.pallas import tpu_sc as plsc

N_NODES = 10000
N_EDGES = 320000
D_NODE = 128
D_EDGE = 16
D_OUT = 128

NC = 2
NS = 16
NW = NC * NS
CH = 128
NCHUNKS = N_EDGES // CH
CH_PER_W = NCHUNKS // NW
CH_EXTRA = NCHUNKS - CH_PER_W * NW


def _node_proj_body(x_ref, w_ref, b_ref, p_ref, q_ref):
    x = x_ref[...]
    w1 = w_ref[0:D_NODE, :]
    w2 = w_ref[D_NODE:2 * D_NODE, :]
    p_ref[...] = jnp.dot(x, w1, preferred_element_type=jnp.float32) + b_ref[...]
    q_ref[...] = jnp.dot(x, w2, preferred_element_type=jnp.float32)


def _node_proj(x, W, b):
    return pl.pallas_call(
        _node_proj_body,
        out_shape=(
            jax.ShapeDtypeStruct((N_NODES, D_OUT), jnp.float32),
            jax.ShapeDtypeStruct((N_NODES, D_OUT), jnp.float32),
        ),
    )(x, W, b.reshape(1, D_OUT))


_EBLK = 8000


def _edge_proj_body(ea_ref, w_ref, e_ref):
    w3 = w_ref[2 * D_NODE:, :]
    e_ref[...] = jnp.dot(ea_ref[...], w3, preferred_element_type=jnp.float32)


def _edge_proj(edge_attr, W):
    grid = (N_EDGES // _EBLK,)
    return pl.pallas_call(
        _edge_proj_body,
        grid=grid,
        in_specs=[
            pl.BlockSpec((_EBLK, D_EDGE), lambda i: (i, 0)),
            pl.BlockSpec((W.shape[0], D_OUT), lambda i: (0, 0)),
        ],
        out_specs=pl.BlockSpec((_EBLK, D_OUT), lambda i: (i, 0)),
        out_shape=jax.ShapeDtypeStruct((N_EDGES, D_OUT), jnp.float32),
    )(edge_attr, W)


NPAIR = CH_PER_W // 2


def _sc_body(src_ref, dst_ref, p_ref, q_ref, e_ref, out_ref,
             idx_s, idx_d, pbuf, qbuf, ebuf, sem_g0, sem_g1, sem_w0, sem_w1):
    sem_g = (sem_g0, sem_g1)
    sem_w = (sem_w0, sem_w1)
    c = lax.axis_index("c")
    s = lax.axis_index("s")
    wid = s * NC + c
    start_ck = wid * CH_PER_W + jnp.minimum(wid, CH_EXTRA)

    def issue(b, ck):
        base = ck * CH
        pltpu.sync_copy(src_ref.at[pl.ds(base, CH)], idx_s.at[b])
        pltpu.sync_copy(dst_ref.at[pl.ds(base, CH)], idx_d.at[b])
        pltpu.async_copy(p_ref.at[idx_s.at[b]], pbuf.at[b], sem_g[b])
        pltpu.async_copy(q_ref.at[idx_d.at[b]], qbuf.at[b], sem_g[b])
        pltpu.async_copy(e_ref.at[pl.ds(base, CH)], ebuf.at[b], sem_g[b])

    def wait_in(b, ck):
        base = ck * CH
        pltpu.make_async_copy(p_ref.at[idx_s.at[b]], pbuf.at[b], sem_g[b]).wait()
        pltpu.make_async_copy(q_ref.at[idx_d.at[b]], qbuf.at[b], sem_g[b]).wait()
        pltpu.make_async_copy(e_ref.at[pl.ds(base, CH)], ebuf.at[b], sem_g[b]).wait()

    def compute(b):
        def row_body(r, rcarry):
            for j in range(D_OUT // 16):
                sl = pl.ds(j * 16, 16)
                plsc.addupdate(ebuf.at[b, r, sl], pbuf[b, r, sl] + qbuf[b, r, sl])
            return rcarry
        lax.fori_loop(0, CH, row_body, 0)

    def issue_out(b, ck):
        pltpu.async_copy(ebuf.at[b], out_ref.at[pl.ds(ck * CH, CH)], sem_w[b])

    def wait_out(b, ck):
        pltpu.make_async_copy(
            ebuf.at[b], out_ref.at[pl.ds(ck * CH, CH)], sem_w[b]).wait()

    def pair_body(i, carry):
        k0 = start_ck + 2 * i

        @pl.when(i > 0)
        def _():
            wait_out(1, k0 - 1)

        issue(1, k0 + 1)
        wait_in(0, k0)
        compute(0)
        issue_out(0, k0)
        wait_in(1, k0 + 1)
        compute(1)

        @pl.when(i < NPAIR - 1)
        def _():
            wait_out(0, k0)
            issue(0, k0 + 2)

        issue_out(1, k0 + 1)
        return carry

    issue(0, start_ck)
    lax.fori_loop(0, NPAIR, pair_body, 0)
    wait_out(0, start_ck + CH_PER_W - 2)
    wait_out(1, start_ck + CH_PER_W - 1)

    @pl.when(wid < CH_EXTRA)
    def _():
        ck = start_ck + CH_PER_W
        issue(0, ck)
        wait_in(0, ck)
        compute(0)
        issue_out(0, ck)
        wait_out(0, ck)


def _sc_gather_add(src, dst, P, Q, E):
    mesh = plsc.VectorSubcoreMesh(
        core_axis_name="c", subcore_axis_name="s", num_cores=NC, num_subcores=NS)
    k = functools.partial(
        pl.kernel,
        mesh=mesh,
        out_type=jax.ShapeDtypeStruct((N_EDGES, D_OUT), jnp.float32),
        scratch_types=[
            pltpu.VMEM((2, CH), jnp.int32),
            pltpu.VMEM((2, CH), jnp.int32),
            pltpu.VMEM((2, CH, D_OUT), jnp.float32),
            pltpu.VMEM((2, CH, D_OUT), jnp.float32),
            pltpu.VMEM((2, CH, D_OUT), jnp.float32),
            pltpu.SemaphoreType.DMA,
            pltpu.SemaphoreType.DMA,
            pltpu.SemaphoreType.DMA,
            pltpu.SemaphoreType.DMA,
        ],
    )(_sc_body)
    return k(src, dst, P, Q, E)


def kernel(x, edge_index, edge_attr, W, b):
    P, Q = _node_proj(x, W, b)
    E = _edge_proj(edge_attr, W)
    return _sc_gather_add(edge_index[0], edge_index[1], P, Q, E)


# speedup vs baseline: 3.5121x; 1.0042x over previous
import functools

import jax
import jax.numpy as jnp
from jax import lax
from jax.experimental import pallas as pl
from jax.experimental.pallas import tpu as pltpu
from jax.experimental.pallas import tpu_sc as plsc

N_NODES = 10000
N_EDGES = 320000
D_NODE = 128
D_EDGE = 16
D_OUT = 128

NC = 2
NS = 16
NW = NC * NS
CH = 128
NCHUNKS = N_EDGES // CH
CH_PER_W = NCHUNKS // NW
CH_EXTRA = NCHUNKS - CH_PER_W * NW


def _node_proj_body(x_ref, w_ref, b_ref, p_ref, q_ref):
    x = x_ref[...]
    w1 = w_ref[0:D_NODE, :]
    w2 = w_ref[D_NODE:2 * D_NODE, :]
    p_ref[...] = jnp.dot(x, w1, preferred_element_type=jnp.float32) + b_ref[...]
    q_ref[...] = jnp.dot(x, w2, preferred_element_type=jnp.float32)


def _node_proj(x, W, b):
    return pl.pallas_call(
        _node_proj_body,
        out_shape=(
            jax.ShapeDtypeStruct((N_NODES, D_OUT), jnp.float32),
            jax.ShapeDtypeStruct((N_NODES, D_OUT), jnp.float32),
        ),
    )(x, W, b.reshape(1, D_OUT))


_EBLK = 16000


def _edge_proj_body(ea_ref, w_ref, e_ref):
    w3 = w_ref[2 * D_NODE:, :]
    e_ref[...] = jnp.dot(ea_ref[...], w3, preferred_element_type=jnp.float32)


def _edge_proj(edge_attr, W):
    grid = (N_EDGES // _EBLK,)
    return pl.pallas_call(
        _edge_proj_body,
        grid=grid,
        in_specs=[
            pl.BlockSpec((_EBLK, D_EDGE), lambda i: (i, 0)),
            pl.BlockSpec((W.shape[0], D_OUT), lambda i: (0, 0)),
        ],
        out_specs=pl.BlockSpec((_EBLK, D_OUT), lambda i: (i, 0)),
        out_shape=jax.ShapeDtypeStruct((N_EDGES, D_OUT), jnp.float32),
    )(edge_attr, W)


NPAIR = CH_PER_W // 2


def _sc_body(src_ref, dst_ref, p_ref, q_ref, e_ref, out_ref,
             idx_s, idx_d, pbuf, qbuf, ebuf, sem_g0, sem_g1, sem_w0, sem_w1):
    sem_g = (sem_g0, sem_g1)
    sem_w = (sem_w0, sem_w1)
    c = lax.axis_index("c")
    s = lax.axis_index("s")
    wid = s * NC + c
    start_ck = wid * CH_PER_W + jnp.minimum(wid, CH_EXTRA)

    def issue(b, ck):
        base = ck * CH
        pltpu.sync_copy(src_ref.at[pl.ds(base, CH)], idx_s.at[b])
        pltpu.sync_copy(dst_ref.at[pl.ds(base, CH)], idx_d.at[b])
        pltpu.async_copy(p_ref.at[idx_s.at[b]], pbuf.at[b], sem_g[b])
        pltpu.async_copy(q_ref.at[idx_d.at[b]], qbuf.at[b], sem_g[b])
        pltpu.async_copy(e_ref.at[pl.ds(base, CH)], ebuf.at[b], sem_g[b])

    def wait_in(b, ck):
        base = ck * CH
        pltpu.make_async_copy(p_ref.at[idx_s.at[b]], pbuf.at[b], sem_g[b]).wait()
        pltpu.make_async_copy(q_ref.at[idx_d.at[b]], qbuf.at[b], sem_g[b]).wait()
        pltpu.make_async_copy(e_ref.at[pl.ds(base, CH)], ebuf.at[b], sem_g[b]).wait()

    def compute(b):
        def row_body(r, rcarry):
            for j in range(D_OUT // 16):
                sl = pl.ds(j * 16, 16)
                plsc.addupdate(ebuf.at[b, r, sl], pbuf[b, r, sl] + qbuf[b, r, sl])
            return rcarry
        lax.fori_loop(0, CH, row_body, 0)

    def issue_out(b, ck):
        pltpu.async_copy(ebuf.at[b], out_ref.at[pl.ds(ck * CH, CH)], sem_w[b])

    def wait_out(b, ck):
        pltpu.make_async_copy(
            ebuf.at[b], out_ref.at[pl.ds(ck * CH, CH)], sem_w[b]).wait()

    def pair_body(i, carry):
        k0 = start_ck + 2 * i

        @pl.when(i > 0)
        def _():
            wait_out(1, k0 - 1)

        issue(1, k0 + 1)
        wait_in(0, k0)
        compute(0)
        issue_out(0, k0)
        wait_in(1, k0 + 1)
        compute(1)

        @pl.when(i < NPAIR - 1)
        def _():
            wait_out(0, k0)
            issue(0, k0 + 2)

        issue_out(1, k0 + 1)
        return carry

    issue(0, start_ck)
    lax.fori_loop(0, NPAIR, pair_body, 0)
    wait_out(0, start_ck + CH_PER_W - 2)
    wait_out(1, start_ck + CH_PER_W - 1)

    @pl.when(wid < CH_EXTRA)
    def _():
        ck = start_ck + CH_PER_W
        issue(0, ck)
        wait_in(0, ck)
        compute(0)
        issue_out(0, ck)
        wait_out(0, ck)


def _sc_gather_add(src, dst, P, Q, E):
    mesh = plsc.VectorSubcoreMesh(
        core_axis_name="c", subcore_axis_name="s", num_cores=NC, num_subcores=NS)
    k = functools.partial(
        pl.kernel,
        mesh=mesh,
        out_type=jax.ShapeDtypeStruct((N_EDGES, D_OUT), jnp.float32),
        scratch_types=[
            pltpu.VMEM((2, CH), jnp.int32),
            pltpu.VMEM((2, CH), jnp.int32),
            pltpu.VMEM((2, CH, D_OUT), jnp.float32),
            pltpu.VMEM((2, CH, D_OUT), jnp.float32),
            pltpu.VMEM((2, CH, D_OUT), jnp.float32),
            pltpu.SemaphoreType.DMA,
            pltpu.SemaphoreType.DMA,
            pltpu.SemaphoreType.DMA,
            pltpu.SemaphoreType.DMA,
        ],
    )(_sc_body)
    return k(src, dst, P, Q, E)


def kernel(x, edge_index, edge_attr, W, b):
    P, Q = _node_proj(x, W, b)
    E = _edge_proj(edge_attr, W)
    return _sc_gather_add(edge_index[0], edge_index[1], P, Q, E)
